# baseline (speedup 1.0000x reference)
import functools

import jax
import jax.numpy as jnp
from jax import lax
from jax.experimental import pallas as pl
from jax.experimental.pallas import tpu as pltpu
from jax.experimental.pallas import tpu_sc as plsc

MEMBERS = 32
NSEL = 40
TOPK_N = 32
ROWS_PER_STEP = 128


def _topk_body(x_ref, idx_ref, cs_ref):
    R = x_ref.shape[0]
    NM = x_ref.shape[1]
    G = NM // MEMBERS
    x = x_ref[...]
    xr = x.reshape(R, MEMBERS, G)

    gmax = jnp.max(xr, axis=1)

    lane_f = lax.broadcasted_iota(jnp.int32, (R, G), 1).astype(jnp.float32)
    neg = jnp.float32(-1e30)

    cur = gmax
    picks = []
    for _ in range(NSEL):
        m = jnp.max(cur, axis=1, keepdims=True)
        ji = jnp.min(jnp.where(cur == m, lane_f, jnp.float32(G)),
                     axis=1, keepdims=True)
        picks.append(ji)
        cur = jnp.where(lane_f == ji, neg, cur)
    jstar = jnp.concatenate(picks, axis=1)

    jg = lax.broadcasted_iota(jnp.int32, (G, NSEL), 0)
    jstar_i = jstar.astype(jnp.int32)
    QS = MEMBERS // 8
    cands = []
    for r in range(R):
        ot_r = (jg == jstar_i[r][None, :]).astype(jnp.float32)
        c = lax.dot_general(xr[r], ot_r, (((1,), (0,)), ((), ())),
                            precision=lax.Precision.HIGHEST,
                            preferred_element_type=jnp.float32)
        crow = jnp.concatenate([c[q * 8:(q + 1) * 8] for q in range(QS)],
                               axis=1)
        cands.append(crow[None])
    cand = jnp.concatenate(cands, axis=0)

    W = QS * NSEL
    l_io = lax.broadcasted_iota(jnp.int32, (R, 8, W), 2)
    sp_io = lax.broadcasted_iota(jnp.int32, (R, 8, W), 1)
    s_idx = (l_io // NSEL) * 8 + sp_io
    jre = jnp.concatenate(
        [jnp.broadcast_to(jstar[:, None, :], (R, 8, NSEL))] * QS, axis=2)
    gi = s_idx.astype(jnp.float32) * jnp.float32(G) + jre
    p = jax.nn.sigmoid(cand)

    big = jnp.float32(1e9)
    out_idx = []
    out_val = []
    curp = p
    for _ in range(TOPK_N):
        m2 = jnp.max(jnp.max(curp, axis=2), axis=1)
        m2 = m2[:, None, None]
        tied = curp == m2
        mi = jnp.min(jnp.min(jnp.where(tied, gi, big), axis=2), axis=1)
        mi = mi[:, None, None]
        out_idx.append(mi[:, :, 0])
        out_val.append(m2[:, :, 0])
        curp = jnp.where(gi == mi, -1.0, curp)
    idxs = jnp.concatenate(out_idx, axis=1).astype(jnp.int32)
    vals = jnp.concatenate(out_val, axis=1)
    idx_ref[...] = idxs

    li = lax.broadcasted_iota(jnp.int32, (TOPK_N, TOPK_N * MEMBERS), 1)
    ti = lax.broadcasted_iota(jnp.int32, (TOPK_N, TOPK_N * MEMBERS), 0)
    E = (li // MEMBERS == ti).astype(jnp.float32)
    cs_ref[...] = lax.dot_general(vals, E, (((1,), (0,)), ((), ())),
                                  precision=lax.Precision.HIGHEST,
                                  preferred_element_type=jnp.float32)


def _run_topk(scores):
    B, NM = scores.shape
    R = ROWS_PER_STEP
    return pl.pallas_call(
        _topk_body,
        grid=(B // R,),
        in_specs=[pl.BlockSpec((R, NM), lambda i: (i, 0))],
        out_specs=[
            pl.BlockSpec((R, TOPK_N), lambda i: (i, 0)),
            pl.BlockSpec((R, TOPK_N * MEMBERS), lambda i: (i, 0)),
        ],
        out_shape=[
            jax.ShapeDtypeStruct((B, TOPK_N), jnp.int32),
            jax.ShapeDtypeStruct((B, TOPK_N * MEMBERS), jnp.float32),
        ],
    )(scores)


def _sc_gather(table, idx_flat):
    n = idx_flat.shape[0]
    d = table.shape[1]
    nw = 32
    bpw = n // nw
    mesh = plsc.VectorSubcoreMesh(core_axis_name="c", subcore_axis_name="s")

    @functools.partial(
        pl.kernel,
        mesh=mesh,
        compiler_params=pltpu.CompilerParams(use_tc_tiling_on_sc=False),
        out_type=jax.ShapeDtypeStruct((n, d), jnp.int32),
        scratch_types=[
            pltpu.VMEM((bpw,), jnp.int32),
            pltpu.VMEM((bpw, d), jnp.int32),
            pltpu.SemaphoreType.DMA,
        ],
    )
    def k(table_hbm, idx_hbm, out_hbm, idx_v, rows_v, sem):
        wid = lax.axis_index("s") * 2 + lax.axis_index("c")
        base = wid * bpw
        pltpu.sync_copy(idx_hbm.at[pl.ds(base, bpw)], idx_v)
        pltpu.async_copy(table_hbm.at[idx_v], rows_v, sem).wait()
        pltpu.sync_copy(rows_v, out_hbm.at[pl.ds(base, bpw)])

    return k(table, idx_flat)


def kernel(scores, label_clusters, k):
    b = scores.shape[0]
    idxs, cscores = _run_topk(scores)
    idxs = idxs + (k - k)
    rows = _sc_gather(label_clusters, idxs.reshape(-1))
    candidates = rows.reshape(b, -1)
    return idxs, candidates, cscores


# speedup vs baseline: 1.1232x; 1.1232x over previous
import functools

import jax
import jax.numpy as jnp
from jax import lax
from jax.experimental import pallas as pl
from jax.experimental.pallas import tpu as pltpu
from jax.experimental.pallas import tpu_sc as plsc

MEMBERS = 32
NSEL = 40
TOPK_N = 32
ROWS_PER_STEP = 64


def _topk_body(x_ref, idx_ref, cs_ref):
    R = x_ref.shape[0]
    NM = x_ref.shape[1]
    G = NM // MEMBERS
    x = x_ref[...]
    xr = x.reshape(R, MEMBERS, G)

    gmax = jnp.max(xr, axis=1)

    lane_f = lax.broadcasted_iota(jnp.int32, (R, G), 1).astype(jnp.float32)
    neg = jnp.float32(-1e30)

    cur = gmax
    picks = []
    for _ in range(NSEL):
        m = jnp.max(cur, axis=1, keepdims=True)
        ji = jnp.min(jnp.where(cur == m, lane_f, jnp.float32(G)),
                     axis=1, keepdims=True)
        picks.append(ji)
        cur = jnp.where(lane_f == ji, neg, cur)
    jstar = jnp.concatenate(picks, axis=1)

    jg = lax.broadcasted_iota(jnp.int32, (G, NSEL), 0)
    jstar_i = jstar.astype(jnp.int32)
    QS = MEMBERS // 8
    cands = []
    for r in range(R):
        ot_r = (jg == jstar_i[r][None, :]).astype(jnp.float32)
        c = lax.dot_general(xr[r], ot_r, (((1,), (0,)), ((), ())),
                            precision=lax.Precision.HIGHEST,
                            preferred_element_type=jnp.float32)
        crow = jnp.concatenate([c[q * 8:(q + 1) * 8] for q in range(QS)],
                               axis=1)
        cands.append(crow[None])
    cand = jnp.concatenate(cands, axis=0)

    W = QS * NSEL
    l_io = lax.broadcasted_iota(jnp.int32, (R, 8, W), 2)
    sp_io = lax.broadcasted_iota(jnp.int32, (R, 8, W), 1)
    s_idx = (l_io // NSEL) * 8 + sp_io
    jre = jnp.concatenate(
        [jnp.broadcast_to(jstar[:, None, :], (R, 8, NSEL))] * QS, axis=2)
    gi = s_idx.astype(jnp.float32) * jnp.float32(G) + jre
    p = jax.nn.sigmoid(cand)

    big = jnp.float32(1e9)
    out_idx = []
    out_val = []
    curp = p
    for _ in range(TOPK_N):
        m2 = jnp.max(jnp.max(curp, axis=2), axis=1)
        m2 = m2[:, None, None]
        tied = curp == m2
        mi = jnp.min(jnp.min(jnp.where(tied, gi, big), axis=2), axis=1)
        mi = mi[:, None, None]
        out_idx.append(mi[:, :, 0])
        out_val.append(m2[:, :, 0])
        curp = jnp.where(gi == mi, -1.0, curp)
    idxs = jnp.concatenate(out_idx, axis=1).astype(jnp.int32)
    vals = jnp.concatenate(out_val, axis=1)
    idx_ref[...] = idxs

    li = lax.broadcasted_iota(jnp.int32, (TOPK_N, TOPK_N * MEMBERS), 1)
    ti = lax.broadcasted_iota(jnp.int32, (TOPK_N, TOPK_N * MEMBERS), 0)
    E = (li // MEMBERS == ti).astype(jnp.float32)
    cs_ref[...] = lax.dot_general(vals, E, (((1,), (0,)), ((), ())),
                                  precision=lax.Precision.HIGHEST,
                                  preferred_element_type=jnp.float32)


def _run_topk(scores):
    B, NM = scores.shape
    R = ROWS_PER_STEP
    return pl.pallas_call(
        _topk_body,
        grid=(B // R,),
        in_specs=[pl.BlockSpec((R, NM), lambda i: (i, 0))],
        out_specs=[
            pl.BlockSpec((R, TOPK_N), lambda i: (i, 0)),
            pl.BlockSpec((R, TOPK_N * MEMBERS), lambda i: (i, 0)),
        ],
        out_shape=[
            jax.ShapeDtypeStruct((B, TOPK_N), jnp.int32),
            jax.ShapeDtypeStruct((B, TOPK_N * MEMBERS), jnp.float32),
        ],
    )(scores)


def _sc_gather(table, idx_flat):
    n = idx_flat.shape[0]
    d = table.shape[1]
    nw = 32
    bpw = n // nw
    mesh = plsc.VectorSubcoreMesh(core_axis_name="c", subcore_axis_name="s")

    @functools.partial(
        pl.kernel,
        mesh=mesh,
        compiler_params=pltpu.CompilerParams(use_tc_tiling_on_sc=False),
        out_type=jax.ShapeDtypeStruct((n, d), jnp.int32),
        scratch_types=[
            pltpu.VMEM((bpw,), jnp.int32),
            pltpu.VMEM((bpw, d), jnp.int32),
            pltpu.SemaphoreType.DMA,
        ],
    )
    def k(table_hbm, idx_hbm, out_hbm, idx_v, rows_v, sem):
        wid = lax.axis_index("s") * 2 + lax.axis_index("c")
        base = wid * bpw
        pltpu.sync_copy(idx_hbm.at[pl.ds(base, bpw)], idx_v)
        pltpu.async_copy(table_hbm.at[idx_v], rows_v, sem).wait()
        pltpu.sync_copy(rows_v, out_hbm.at[pl.ds(base, bpw)])

    return k(table, idx_flat)


def kernel(scores, label_clusters, k):
    b = scores.shape[0]
    idxs, cscores = _run_topk(scores)
    idxs = idxs + (k - k)
    rows = _sc_gather(label_clusters, idxs.reshape(-1))
    candidates = rows.reshape(b, -1)
    return idxs, candidates, cscores


# speedup vs baseline: 1.1382x; 1.0133x over previous
import functools

import jax
import jax.numpy as jnp
from jax import lax
from jax.experimental import pallas as pl
from jax.experimental.pallas import tpu as pltpu
from jax.experimental.pallas import tpu_sc as plsc

MEMBERS = 32
NSEL = 36
TOPK_N = 32
ROWS_PER_STEP = 64


def _topk_body(x_ref, idx_ref, cs_ref):
    R = x_ref.shape[0]
    NM = x_ref.shape[1]
    G = NM // MEMBERS
    x = x_ref[...]
    xr = x.reshape(R, MEMBERS, G)

    gmax = jnp.max(xr, axis=1)

    lane_f = lax.broadcasted_iota(jnp.int32, (R, G), 1).astype(jnp.float32)
    neg = jnp.float32(-1e30)

    cur = gmax
    picks = []
    for _ in range(NSEL):
        m = jnp.max(cur, axis=1, keepdims=True)
        ji = jnp.min(jnp.where(cur == m, lane_f, jnp.float32(G)),
                     axis=1, keepdims=True)
        picks.append(ji)
        cur = jnp.where(lane_f == ji, neg, cur)
    jstar = jnp.concatenate(picks, axis=1)

    jg = lax.broadcasted_iota(jnp.int32, (G, NSEL), 0)
    jstar_i = jstar.astype(jnp.int32)
    QS = MEMBERS // 8
    cands = []
    for r in range(R):
        ot_r = (jg == jstar_i[r][None, :]).astype(jnp.float32)
        c = lax.dot_general(xr[r], ot_r, (((1,), (0,)), ((), ())),
                            precision=lax.Precision.HIGHEST,
                            preferred_element_type=jnp.float32)
        crow = jnp.concatenate([c[q * 8:(q + 1) * 8] for q in range(QS)],
                               axis=1)
        cands.append(crow[None])
    cand = jnp.concatenate(cands, axis=0)

    W = QS * NSEL
    l_io = lax.broadcasted_iota(jnp.int32, (R, 8, W), 2)
    sp_io = lax.broadcasted_iota(jnp.int32, (R, 8, W), 1)
    s_idx = (l_io // NSEL) * 8 + sp_io
    jre = jnp.concatenate(
        [jnp.broadcast_to(jstar[:, None, :], (R, 8, NSEL))] * QS, axis=2)
    gi = s_idx.astype(jnp.float32) * jnp.float32(G) + jre
    p = jax.nn.sigmoid(cand)

    big = jnp.float32(1e9)
    out_idx = []
    out_val = []
    curp = p
    for _ in range(TOPK_N):
        m2 = jnp.max(jnp.max(curp, axis=2), axis=1)
        m2 = m2[:, None, None]
        tied = curp == m2
        mi = jnp.min(jnp.min(jnp.where(tied, gi, big), axis=2), axis=1)
        mi = mi[:, None, None]
        out_idx.append(mi[:, :, 0])
        out_val.append(m2[:, :, 0])
        curp = jnp.where(gi == mi, -1.0, curp)
    idxs = jnp.concatenate(out_idx, axis=1).astype(jnp.int32)
    vals = jnp.concatenate(out_val, axis=1)
    idx_ref[...] = idxs

    li = lax.broadcasted_iota(jnp.int32, (TOPK_N, TOPK_N * MEMBERS), 1)
    ti = lax.broadcasted_iota(jnp.int32, (TOPK_N, TOPK_N * MEMBERS), 0)
    E = (li // MEMBERS == ti).astype(jnp.float32)
    cs_ref[...] = lax.dot_general(vals, E, (((1,), (0,)), ((), ())),
                                  precision=lax.Precision.HIGHEST,
                                  preferred_element_type=jnp.float32)


def _run_topk(scores):
    B, NM = scores.shape
    R = ROWS_PER_STEP
    return pl.pallas_call(
        _topk_body,
        grid=(B // R,),
        in_specs=[pl.BlockSpec((R, NM), lambda i: (i, 0))],
        out_specs=[
            pl.BlockSpec((R, TOPK_N), lambda i: (i, 0)),
            pl.BlockSpec((R, TOPK_N * MEMBERS), lambda i: (i, 0)),
        ],
        out_shape=[
            jax.ShapeDtypeStruct((B, TOPK_N), jnp.int32),
            jax.ShapeDtypeStruct((B, TOPK_N * MEMBERS), jnp.float32),
        ],
    )(scores)


def _sc_gather(table, idx_flat):
    n = idx_flat.shape[0]
    d = table.shape[1]
    nw = 32
    bpw = n // nw
    mesh = plsc.VectorSubcoreMesh(core_axis_name="c", subcore_axis_name="s")

    @functools.partial(
        pl.kernel,
        mesh=mesh,
        compiler_params=pltpu.CompilerParams(use_tc_tiling_on_sc=False),
        out_type=jax.ShapeDtypeStruct((n, d), jnp.int32),
        scratch_types=[
            pltpu.VMEM((bpw,), jnp.int32),
            pltpu.VMEM((bpw, d), jnp.int32),
            pltpu.SemaphoreType.DMA,
        ],
    )
    def k(table_hbm, idx_hbm, out_hbm, idx_v, rows_v, sem):
        wid = lax.axis_index("s") * 2 + lax.axis_index("c")
        base = wid * bpw
        pltpu.sync_copy(idx_hbm.at[pl.ds(base, bpw)], idx_v)
        pltpu.async_copy(table_hbm.at[idx_v], rows_v, sem).wait()
        pltpu.sync_copy(rows_v, out_hbm.at[pl.ds(base, bpw)])

    return k(table, idx_flat)


def kernel(scores, label_clusters, k):
    b = scores.shape[0]
    idxs, cscores = _run_topk(scores)
    idxs = idxs + (k - k)
    rows = _sc_gather(label_clusters, idxs.reshape(-1))
    candidates = rows.reshape(b, -1)
    return idxs, candidates, cscores
